# Initial kernel scaffold; baseline (speedup 1.0000x reference)
import functools

import jax
import jax.numpy as jnp
from jax import lax
from jax.experimental import pallas as pl
from jax.experimental.pallas import tpu as pltpu
from jax.experimental.pallas import tpu_sc as plsc

NUM_WORDS = 1000000
DIM_EMBED = 32
NUM_CLASSES = 2
BATCH = 4096
SEQ = 200

NW = 32
CHUNK = 100
CHUNKS_PER_ROW = SEQ // CHUNK
ROWS_PER_W = BATCH // NW
CHUNKS_PER_W = ROWS_PER_W * CHUNKS_PER_ROW
HALF = 16


def _sc_gather_sums(x_flat, table):
    mesh = plsc.VectorSubcoreMesh(core_axis_name="c", subcore_axis_name="s")

    @functools.partial(
        pl.kernel,
        out_type=jax.ShapeDtypeStruct((BATCH * CHUNKS_PER_ROW, DIM_EMBED),
                                      jnp.float32),
        mesh=mesh,
        scratch_types=[
            pltpu.VMEM((CHUNKS_PER_W, CHUNK), jnp.int32),
            pltpu.VMEM((CHUNK, DIM_EMBED), jnp.float32),
            pltpu.VMEM((CHUNKS_PER_W, DIM_EMBED), jnp.float32),
            pltpu.SemaphoreType.DMA,
        ],
    )
    def k(x_hbm, table_hbm, out_hbm, idx_v, rows_v, sums_v, sem):
        wid = lax.axis_index("s") * 2 + lax.axis_index("c")
        base = wid * CHUNKS_PER_W
        pltpu.sync_copy(x_hbm.at[pl.ds(base, CHUNKS_PER_W)], idx_v)

        def chunk_body(t, _):
            cp = pltpu.async_copy(table_hbm.at[idx_v.at[t]], rows_v, sem)
            cp.wait()
            accA = [jnp.zeros((HALF,), jnp.float32) for _ in range(4)]
            accB = [jnp.zeros((HALF,), jnp.float32) for _ in range(4)]
            for j in range(CHUNK):
                accA[j % 4] = accA[j % 4] + rows_v[j, pl.ds(0, HALF)]
                accB[j % 4] = accB[j % 4] + rows_v[j, pl.ds(HALF, HALF)]
            sums_v[t, pl.ds(0, HALF)] = (accA[0] + accA[1]) + (accA[2] + accA[3])
            sums_v[t, pl.ds(HALF, HALF)] = (accB[0] + accB[1]) + (accB[2] + accB[3])
            return 0

        lax.fori_loop(0, CHUNKS_PER_W, chunk_body, 0)
        pltpu.sync_copy(sums_v, out_hbm.at[pl.ds(base, CHUNKS_PER_W)])

    return k(x_flat, table)


def _tc_fc(sums2, wt, bias):
    def body(s_ref, w_ref, b_ref, o_ref):
        s = s_ref[:]
        avg = (s[:, :DIM_EMBED] + s[:, DIM_EMBED:]) * (1.0 / SEQ)
        o_ref[:] = (
            jnp.dot(avg, w_ref[:], preferred_element_type=jnp.float32)
            + b_ref[:]
        )

    return pl.pallas_call(
        body,
        out_shape=jax.ShapeDtypeStruct((BATCH, NUM_CLASSES), jnp.float32),
    )(sums2, wt, bias)


def kernel(x, embedding_table, fc_weight, fc_bias):
    x_flat = jnp.reshape(x.astype(jnp.int32), (-1, CHUNK))
    sums = _sc_gather_sums(x_flat, embedding_table)
    sums2 = jnp.reshape(sums, (BATCH, 2 * DIM_EMBED))
    out = _tc_fc(sums2, fc_weight.T, jnp.reshape(fc_bias, (1, NUM_CLASSES)))
    return out


# speedup vs baseline: 1.8002x; 1.8002x over previous
import functools

import jax
import jax.numpy as jnp
from jax import lax
from jax.experimental import pallas as pl
from jax.experimental.pallas import tpu as pltpu
from jax.experimental.pallas import tpu_sc as plsc

NUM_WORDS = 1000000
DIM_EMBED = 32
NUM_CLASSES = 2
BATCH = 4096
SEQ = 200

NW = 32
CHUNK = 100
CHUNKS_PER_ROW = SEQ // CHUNK
ROWS_PER_W = BATCH // NW
CHUNKS_PER_W = ROWS_PER_W * CHUNKS_PER_ROW
HALF = 16


def _sc_gather_sums(x_flat, table):
    mesh = plsc.VectorSubcoreMesh(core_axis_name="c", subcore_axis_name="s")

    @functools.partial(
        pl.kernel,
        out_type=jax.ShapeDtypeStruct((BATCH * CHUNKS_PER_ROW, DIM_EMBED),
                                      jnp.float32),
        mesh=mesh,
        scratch_types=[
            pltpu.VMEM((CHUNKS_PER_W, CHUNK), jnp.int32),
            pltpu.VMEM((CHUNK, DIM_EMBED), jnp.float32),
            pltpu.VMEM((CHUNKS_PER_W, DIM_EMBED), jnp.float32),
            pltpu.SemaphoreType.DMA,
        ],
        compiler_params=pltpu.CompilerParams(use_tc_tiling_on_sc=False),
    )
    def k(x_hbm, table_hbm, out_hbm, idx_v, rows_v, sums_v, sem):
        wid = lax.axis_index("s") * 2 + lax.axis_index("c")
        base = wid * CHUNKS_PER_W
        pltpu.sync_copy(x_hbm.at[pl.ds(base, CHUNKS_PER_W)], idx_v)

        def chunk_body(t, _):
            cp = pltpu.async_copy(table_hbm.at[idx_v.at[t]], rows_v, sem)
            cp.wait()
            accA = [jnp.zeros((HALF,), jnp.float32) for _ in range(4)]
            accB = [jnp.zeros((HALF,), jnp.float32) for _ in range(4)]
            for j in range(CHUNK):
                accA[j % 4] = accA[j % 4] + rows_v[j, pl.ds(0, HALF)]
                accB[j % 4] = accB[j % 4] + rows_v[j, pl.ds(HALF, HALF)]
            sums_v[t, pl.ds(0, HALF)] = (accA[0] + accA[1]) + (accA[2] + accA[3])
            sums_v[t, pl.ds(HALF, HALF)] = (accB[0] + accB[1]) + (accB[2] + accB[3])
            return 0

        lax.fori_loop(0, CHUNKS_PER_W, chunk_body, 0)
        pltpu.sync_copy(sums_v, out_hbm.at[pl.ds(base, CHUNKS_PER_W)])

    return k(x_flat, table)


def _tc_fc(sums2, wt, bias):
    def body(s_ref, w_ref, b_ref, o_ref):
        s = s_ref[:]
        avg = (s[:, :DIM_EMBED] + s[:, DIM_EMBED:]) * (1.0 / SEQ)
        o_ref[:] = (
            jnp.dot(avg, w_ref[:], preferred_element_type=jnp.float32)
            + b_ref[:]
        )

    return pl.pallas_call(
        body,
        out_shape=jax.ShapeDtypeStruct((BATCH, NUM_CLASSES), jnp.float32),
    )(sums2, wt, bias)


def kernel(x, embedding_table, fc_weight, fc_bias):
    x_flat = jnp.reshape(x.astype(jnp.int32), (-1, CHUNK))
    sums = _sc_gather_sums(x_flat, embedding_table)
    sums2 = jnp.reshape(sums, (BATCH, 2 * DIM_EMBED))
    out = _tc_fc(sums2, fc_weight.T, jnp.reshape(fc_bias, (1, NUM_CLASSES)))
    return out


# speedup vs baseline: 2.3134x; 1.2851x over previous
import functools

import jax
import jax.numpy as jnp
from jax import lax
from jax.experimental import pallas as pl
from jax.experimental.pallas import tpu as pltpu
from jax.experimental.pallas import tpu_sc as plsc

NUM_WORDS = 1000000
DIM_EMBED = 32
NUM_CLASSES = 2
BATCH = 4096
SEQ = 200

NW = 32
CHUNK = 100
CHUNKS_PER_ROW = SEQ // CHUNK
ROWS_PER_W = BATCH // NW
CHUNKS_PER_W = ROWS_PER_W * CHUNKS_PER_ROW
HALF = 16
NBUF = 4


def _sc_gather_sums(x_flat, table):
    mesh = plsc.VectorSubcoreMesh(core_axis_name="c", subcore_axis_name="s")

    @functools.partial(
        pl.kernel,
        out_type=jax.ShapeDtypeStruct((BATCH * CHUNKS_PER_ROW, DIM_EMBED),
                                      jnp.float32),
        mesh=mesh,
        scratch_types=[
            pltpu.VMEM((CHUNKS_PER_W, CHUNK), jnp.int32),
            pltpu.VMEM((NBUF, CHUNK, DIM_EMBED), jnp.float32),
            pltpu.VMEM((CHUNKS_PER_W, DIM_EMBED), jnp.float32),
            pltpu.SemaphoreType.DMA((NBUF,)),
        ],
        compiler_params=pltpu.CompilerParams(use_tc_tiling_on_sc=False),
    )
    def k(x_hbm, table_hbm, out_hbm, idx_v, rows_v, sums_v, sem):
        wid = lax.axis_index("s") * 2 + lax.axis_index("c")
        base = wid * CHUNKS_PER_W
        pltpu.sync_copy(x_hbm.at[pl.ds(base, CHUNKS_PER_W)], idx_v)

        def gather(t, b):
            pltpu.make_async_copy(
                table_hbm.at[idx_v.at[t]], rows_v.at[b], sem.at[b]
            ).start()

        for b in range(NBUF):
            gather(b, b)

        def group_body(g, _):
            t0 = g * NBUF
            for b in range(NBUF):
                t = t0 + b
                pltpu.make_async_copy(
                    table_hbm.at[idx_v.at[t]], rows_v.at[b], sem.at[b]
                ).wait()
                accA = [jnp.zeros((HALF,), jnp.float32) for _ in range(4)]
                accB = [jnp.zeros((HALF,), jnp.float32) for _ in range(4)]
                for j in range(CHUNK):
                    accA[j % 4] = accA[j % 4] + rows_v[b, j, pl.ds(0, HALF)]
                    accB[j % 4] = accB[j % 4] + rows_v[b, j, pl.ds(HALF, HALF)]
                @pl.when(g < CHUNKS_PER_W // NBUF - 1)
                def _():
                    gather(t + NBUF, b)
                sums_v[t, pl.ds(0, HALF)] = (
                    (accA[0] + accA[1]) + (accA[2] + accA[3]))
                sums_v[t, pl.ds(HALF, HALF)] = (
                    (accB[0] + accB[1]) + (accB[2] + accB[3]))
            return 0

        lax.fori_loop(0, CHUNKS_PER_W // NBUF, group_body, 0)
        pltpu.sync_copy(sums_v, out_hbm.at[pl.ds(base, CHUNKS_PER_W)])

    return k(x_flat, table)


def _tc_fc(sums2, wt, bias):
    def body(s_ref, w_ref, b_ref, o_ref):
        s = s_ref[:]
        avg = (s[:, :DIM_EMBED] + s[:, DIM_EMBED:]) * (1.0 / SEQ)
        o_ref[:] = (
            jnp.dot(avg, w_ref[:], preferred_element_type=jnp.float32)
            + b_ref[:]
        )

    return pl.pallas_call(
        body,
        out_shape=jax.ShapeDtypeStruct((BATCH, NUM_CLASSES), jnp.float32),
    )(sums2, wt, bias)


def kernel(x, embedding_table, fc_weight, fc_bias):
    x_flat = jnp.reshape(x.astype(jnp.int32), (-1, CHUNK))
    sums = _sc_gather_sums(x_flat, embedding_table)
    sums2 = jnp.reshape(sums, (BATCH, 2 * DIM_EMBED))
    out = _tc_fc(sums2, fc_weight.T, jnp.reshape(fc_bias, (1, NUM_CLASSES)))
    return out
